# baseline (speedup 1.0000x reference)
import functools

import jax
import jax.numpy as jnp
from jax import lax
from jax.experimental import pallas as pl
from jax.experimental.pallas import tpu as pltpu
from jax.experimental.pallas import tpu_sc as plsc

DIM = 1024
E = 8
H = 2048
N = 4096
NA = 2 * N
EPAD = 16
NEG = -1e30

BS = 256
NBLK = NA // BS + E
NSLOT = NBLK * BS

NW = 32
TPW = N // NW
SCCH = 32
GCH = 64


def _gelu_exact(u):
    return 0.5 * u * (1.0 + jax.lax.erf(u * (2.0 ** -0.5)))


def _router_body(x_ref, rw_ref, rb_ref, w_ref, i_ref, ws_ref):
    x = x_ref[...]
    logits = jnp.dot(x, rw_ref[...], preferred_element_type=jnp.float32)
    logits = logits + rb_ref[...]
    m = jnp.max(logits, axis=1, keepdims=True)
    ex = jnp.exp(logits - m)
    probs = ex / jnp.sum(ex, axis=1, keepdims=True)
    lane = jax.lax.broadcasted_iota(jnp.int32, probs.shape, 1)
    m1 = jnp.max(probs, axis=1, keepdims=True)
    i1 = jnp.min(jnp.where(probs == m1, lane, 512), axis=1, keepdims=True)
    probs2 = jnp.where(lane == i1, NEG, probs)
    m2 = jnp.max(probs2, axis=1, keepdims=True)
    i2 = jnp.min(jnp.where(probs2 == m2, lane, 512), axis=1, keepdims=True)
    w = jnp.where(lane == 0, m1, 0.0) + jnp.where(lane == 1, m2, 0.0)
    ii = jnp.where(lane == 0, i1, 0) + jnp.where(lane == 1, i2, 0)
    w_ref[...] = w[:, :EPAD]
    i_ref[...] = ii[:, :EPAD]
    tb = m1.shape[0]
    ws_ref[...] = jnp.concatenate(
        [jnp.broadcast_to(m1[:, None], (tb, 1, 16)),
         jnp.broadcast_to(m2[:, None], (tb, 1, 16))], axis=1)


def _meta_body(ej_ref, p_ref, blk_ref):
    ej = ej_ref[...]
    rows, cols = ej.shape
    r1 = jax.lax.broadcasted_iota(jnp.int32, (cols, cols), 0)
    c1 = jax.lax.broadcasted_iota(jnp.int32, (cols, cols), 1)
    U = (r1 <= c1).astype(jnp.float32)
    r2 = jax.lax.broadcasted_iota(jnp.int32, (rows, rows), 0)
    c2 = jax.lax.broadcasted_iota(jnp.int32, (rows, rows), 1)
    S = (c2 < r2).astype(jnp.float32)

    start = jnp.zeros((), jnp.int32)
    p = jnp.zeros(ej.shape, jnp.int32)
    bi = jax.lax.broadcasted_iota(jnp.int32, (8, 128), 1) * BS
    blk = jnp.zeros((8, 128), jnp.int32)
    for e in range(E):
        m = (ej == e).astype(jnp.float32)
        intra = jnp.dot(m, U, preferred_element_type=jnp.float32)
        rowtot = intra[:, cols - 1:cols]
        rowoff = jnp.dot(S, rowtot, preferred_element_type=jnp.float32)
        rank_excl = (intra + rowoff - 1.0).astype(jnp.int32)
        p = p + jnp.where(ej == e, start + rank_excl, 0)
        cnt = jnp.sum(m).astype(jnp.int32)
        cap = ((cnt + BS - 1) // BS) * BS
        blk = blk + jnp.where(
            jnp.logical_and(bi >= start, bi < start + cap), e, 0)
        start = start + cap
    blk = blk + jnp.where(bi >= start, E - 1, 0)
    p_ref[...] = p
    blk_ref[...] = blk


def _sc_scatter_body(x_hbm, pidx_hbm, xg_hbm, idx_v, rows_v, sem):
    wid = lax.axis_index("s") * 2 + lax.axis_index("c")
    pltpu.sync_copy(pidx_hbm.at[wid], idx_v)
    for c in range(TPW // SCCH):
        pltpu.sync_copy(x_hbm.at[pl.ds(wid * TPW + c * SCCH, SCCH)], rows_v)
        cp0 = pltpu.async_copy(rows_v, xg_hbm.at[idx_v.at[2 * c]], sem)
        cp1 = pltpu.async_copy(rows_v, xg_hbm.at[idx_v.at[2 * c + 1]], sem)
        cp0.wait()
        cp1.wait()


def _group_body(blk_ref, xg_ref, w1_ref, b1_ref, w2_ref, b2_ref, y_ref):
    x = xg_ref[...].astype(jnp.bfloat16)
    h = jnp.dot(x, w1_ref[0].astype(jnp.bfloat16),
                preferred_element_type=jnp.float32)
    h = h + b1_ref[0]
    h = _gelu_exact(h).astype(jnp.bfloat16)
    y = jnp.dot(h, w2_ref[0].astype(jnp.bfloat16),
                preferred_element_type=jnp.float32)
    y_ref[...] = y + b2_ref[0]


def _sc_gather_body(y_hbm, gidx_hbm, wj_hbm, yg_hbm, idx_v, rows_v, out_v,
                    w_v, sem):
    wid = lax.axis_index("s") * 2 + lax.axis_index("c")
    pltpu.sync_copy(gidx_hbm.at[wid], idx_v)
    for c in range(NA // NW // GCH):
        pltpu.sync_copy(
            wj_hbm.at[pl.ds(wid * (NA // NW) + c * GCH, GCH)], w_v)
        pltpu.async_copy(y_hbm.at[idx_v.at[c]], rows_v, sem).wait()

        def tok(i2, _):
            w0 = w_v[2 * i2, :]
            w1 = w_v[2 * i2 + 1, :]
            for v in range(DIM // 16):
                sl = pl.ds(16 * v, 16)
                out_v[i2, sl] = (rows_v[2 * i2, sl] * w0
                                 + rows_v[2 * i2 + 1, sl] * w1)
            return 0

        lax.fori_loop(0, GCH // 2, tok, 0)
        pltpu.sync_copy(
            out_v, yg_hbm.at[pl.ds(wid * TPW + c * (GCH // 2), GCH // 2)])


def _shared_body(x_ref, w1_ref, b1_ref, w2_ref, b2_ref, yg_ref, out_ref):
    x = x_ref[...].astype(jnp.bfloat16)
    h = jnp.dot(x, w1_ref[...].astype(jnp.bfloat16),
                preferred_element_type=jnp.float32)
    h = h + b1_ref[...]
    h = _gelu_exact(h).astype(jnp.bfloat16)
    y = jnp.dot(h, w2_ref[...].astype(jnp.bfloat16),
                preferred_element_type=jnp.float32)
    out_ref[...] = y + b2_ref[...] + yg_ref[...]


def kernel(x, router_W, router_b, W1, b1, W2, b2, sW1, sb1, sW2, sb2):
    B, S, dim = x.shape
    xf = x.reshape(N, dim)

    rw = jnp.zeros((dim, 128), jnp.float32).at[:, :E].set(router_W)
    rb = jnp.full((1, 128), NEG, jnp.float32).at[0, :E].set(router_b)
    TB = 1024
    wts, tops, wsplat = pl.pallas_call(
        _router_body,
        grid=(N // TB,),
        in_specs=[
            pl.BlockSpec((TB, dim), lambda t: (t, 0)),
            pl.BlockSpec((dim, 128), lambda t: (0, 0)),
            pl.BlockSpec((1, 128), lambda t: (0, 0)),
        ],
        out_specs=[
            pl.BlockSpec((TB, EPAD), lambda t: (t, 0)),
            pl.BlockSpec((TB, EPAD), lambda t: (t, 0)),
            pl.BlockSpec((TB, 2, 16), lambda t: (t, 0, 0)),
        ],
        out_shape=[
            jax.ShapeDtypeStruct((N, EPAD), jnp.float32),
            jax.ShapeDtypeStruct((N, EPAD), jnp.int32),
            jax.ShapeDtypeStruct((N, 2, 16), jnp.float32),
        ],
    )(xf, rw, rb)

    ej = tops[:, :2].reshape(64, 128)

    p2d, blk2d = pl.pallas_call(
        _meta_body,
        out_shape=[
            jax.ShapeDtypeStruct((64, 128), jnp.int32),
            jax.ShapeDtypeStruct((8, 128), jnp.int32),
        ],
    )(ej)
    p = p2d.reshape(NA)
    blk_expert = blk2d[0, :NBLK]

    pidx = p.reshape(NW, TPW // SCCH, SCCH, 2).transpose(0, 1, 3, 2)
    pidx = pidx.reshape(NW, 2 * (TPW // SCCH), SCCH)
    mesh = plsc.VectorSubcoreMesh(core_axis_name="c", subcore_axis_name="s")
    xg = pl.kernel(
        _sc_scatter_body,
        mesh=mesh,
        out_type=jax.ShapeDtypeStruct((NSLOT, DIM), jnp.float32),
        scratch_types=[
            pltpu.VMEM((2 * (TPW // SCCH), SCCH), jnp.int32),
            pltpu.VMEM((SCCH, DIM), jnp.float32),
            pltpu.SemaphoreType.DMA,
        ],
    )(xf, pidx)

    b1r = b1[:, None]
    b2r = b2[:, None]
    y = pl.pallas_call(
        _group_body,
        grid_spec=pltpu.PrefetchScalarGridSpec(
            num_scalar_prefetch=1,
            grid=(NBLK,),
            in_specs=[
                pl.BlockSpec((BS, DIM), lambda b, s: (b, 0)),
                pl.BlockSpec((1, DIM, H), lambda b, s: (s[b], 0, 0)),
                pl.BlockSpec((1, 1, H), lambda b, s: (s[b], 0, 0)),
                pl.BlockSpec((1, H, DIM), lambda b, s: (s[b], 0, 0)),
                pl.BlockSpec((1, 1, DIM), lambda b, s: (s[b], 0, 0)),
            ],
            out_specs=pl.BlockSpec((BS, DIM), lambda b, s: (b, 0)),
        ),
        out_shape=jax.ShapeDtypeStruct((NSLOT, DIM), jnp.float32),
        compiler_params=pltpu.CompilerParams(
            vmem_limit_bytes=62 * 1024 * 1024),
    )(blk_expert, xg, W1, b1r, W2, b2r)

    gidx = p.reshape(NW, NA // NW // GCH, GCH)
    wj = wsplat.reshape(NA, 16)
    ygw = pl.kernel(
        _sc_gather_body,
        mesh=mesh,
        out_type=jax.ShapeDtypeStruct((N, DIM), jnp.float32),
        scratch_types=[
            pltpu.VMEM((NA // NW // GCH, GCH), jnp.int32),
            pltpu.VMEM((GCH, DIM), jnp.float32),
            pltpu.VMEM((GCH // 2, DIM), jnp.float32),
            pltpu.VMEM((GCH, 16), jnp.float32),
            pltpu.SemaphoreType.DMA,
        ],
    )(y, gidx, wj)

    TBS = 512
    out = pl.pallas_call(
        _shared_body,
        grid=(N // TBS,),
        in_specs=[
            pl.BlockSpec((TBS, dim), lambda t: (t, 0)),
            pl.BlockSpec((dim, H), lambda t: (0, 0)),
            pl.BlockSpec((1, H), lambda t: (0, 0)),
            pl.BlockSpec((H, dim), lambda t: (0, 0)),
            pl.BlockSpec((1, dim), lambda t: (0, 0)),
            pl.BlockSpec((TBS, dim), lambda t: (t, 0)),
        ],
        out_specs=pl.BlockSpec((TBS, dim), lambda t: (t, 0)),
        out_shape=jax.ShapeDtypeStruct((N, dim), jnp.float32),
        compiler_params=pltpu.CompilerParams(
            vmem_limit_bytes=62 * 1024 * 1024),
    )(xf, sW1, sb1[None], sW2, sb2[None], ygw)

    return out.reshape(B, S, dim)


# speedup vs baseline: 1.0645x; 1.0645x over previous
import functools

import jax
import jax.numpy as jnp
from jax import lax
from jax.experimental import pallas as pl
from jax.experimental.pallas import tpu as pltpu
from jax.experimental.pallas import tpu_sc as plsc

DIM = 1024
E = 8
H = 2048
N = 4096
NA = 2 * N
EPAD = 16
NEG = -1e30

BS = 512
NBLK = NA // BS + E
NSLOT = NBLK * BS

NW = 32
TPW = N // NW
SCCH = 32
GCH = 64


def _gelu_exact(u):
    return 0.5 * u * (1.0 + jax.lax.erf(u * (2.0 ** -0.5)))


def _router_body(x_ref, rw_ref, rb_ref, w_ref, i_ref):
    x = x_ref[...]
    logits = jnp.dot(x, rw_ref[...], preferred_element_type=jnp.float32)
    logits = logits + rb_ref[...]
    m = jnp.max(logits, axis=1, keepdims=True)
    ex = jnp.exp(logits - m)
    probs = ex / jnp.sum(ex, axis=1, keepdims=True)
    lane = jax.lax.broadcasted_iota(jnp.int32, probs.shape, 1)
    m1 = jnp.max(probs, axis=1, keepdims=True)
    i1 = jnp.min(jnp.where(probs == m1, lane, 512), axis=1, keepdims=True)
    probs2 = jnp.where(lane == i1, NEG, probs)
    m2 = jnp.max(probs2, axis=1, keepdims=True)
    i2 = jnp.min(jnp.where(probs2 == m2, lane, 512), axis=1, keepdims=True)
    w = jnp.where(lane == 0, m1, 0.0) + jnp.where(lane == 1, m2, 0.0)
    ii = jnp.where(lane == 0, i1, 0) + jnp.where(lane == 1, i2, 0)
    w_ref[...] = w[:, :EPAD]
    i_ref[...] = ii[:, :EPAD]


def _meta_body(ej_ref, p_ref, blk_ref):
    ej = ej_ref[...]
    rows, cols = ej.shape
    r1 = jax.lax.broadcasted_iota(jnp.int32, (cols, cols), 0)
    c1 = jax.lax.broadcasted_iota(jnp.int32, (cols, cols), 1)
    U = (r1 <= c1).astype(jnp.float32)
    r2 = jax.lax.broadcasted_iota(jnp.int32, (rows, rows), 0)
    c2 = jax.lax.broadcasted_iota(jnp.int32, (rows, rows), 1)
    S = (c2 < r2).astype(jnp.float32)

    start = jnp.zeros((), jnp.int32)
    p = jnp.zeros(ej.shape, jnp.int32)
    bi = jax.lax.broadcasted_iota(jnp.int32, (8, 128), 1) * BS
    blk = jnp.zeros((8, 128), jnp.int32)
    for e in range(E):
        m = (ej == e).astype(jnp.float32)
        intra = jnp.dot(m, U, preferred_element_type=jnp.float32)
        rowtot = intra[:, cols - 1:cols]
        rowoff = jnp.dot(S, rowtot, preferred_element_type=jnp.float32)
        rank_excl = (intra + rowoff - 1.0).astype(jnp.int32)
        p = p + jnp.where(ej == e, start + rank_excl, 0)
        cnt = jnp.sum(m).astype(jnp.int32)
        cap = ((cnt + BS - 1) // BS) * BS
        blk = blk + jnp.where(
            jnp.logical_and(bi >= start, bi < start + cap), e, 0)
        start = start + cap
    blk = blk + jnp.where(bi >= start, E - 1, 0)
    p_ref[...] = p
    blk_ref[...] = blk


def _sc_scatter_body(x_hbm, pidx_hbm, xg_hbm, idx_v, rows_v, sem):
    wid = lax.axis_index("s") * 2 + lax.axis_index("c")
    pltpu.sync_copy(pidx_hbm.at[wid], idx_v)
    for c in range(TPW // SCCH):
        pltpu.sync_copy(x_hbm.at[pl.ds(wid * TPW + c * SCCH, SCCH)], rows_v)
        cp0 = pltpu.async_copy(rows_v, xg_hbm.at[idx_v.at[2 * c]], sem)
        cp1 = pltpu.async_copy(rows_v, xg_hbm.at[idx_v.at[2 * c + 1]], sem)
        cp0.wait()
        cp1.wait()


def _group_body(blk_ref, xg_ref, w1_ref, b1_ref, w2_ref, b2_ref, y_ref):
    x = xg_ref[...].astype(jnp.bfloat16)
    h = jnp.dot(x, w1_ref[0].astype(jnp.bfloat16),
                preferred_element_type=jnp.float32)
    h = h + b1_ref[0]
    h = _gelu_exact(h).astype(jnp.bfloat16)
    y = jnp.dot(h, w2_ref[0].astype(jnp.bfloat16),
                preferred_element_type=jnp.float32)
    y_ref[...] = y + b2_ref[0]


def _sc_gather_body(y_hbm, gidx_hbm, yg_hbm, idx_v, rows_v, sem):
    wid = lax.axis_index("s") * 2 + lax.axis_index("c")
    pltpu.sync_copy(gidx_hbm.at[wid], idx_v)
    for c in range(NA // NW // GCH):
        pltpu.async_copy(y_hbm.at[idx_v.at[c]], rows_v, sem).wait()
        pltpu.sync_copy(rows_v, yg_hbm.at[pl.ds(wid * (NA // NW) + c * GCH,
                                                GCH)])


def _shared_body(x_ref, w1_ref, b1_ref, w2_ref, b2_ref, wts_ref, yg_ref,
                 out_ref):
    x = x_ref[...].astype(jnp.bfloat16)
    h = jnp.dot(x, w1_ref[...].astype(jnp.bfloat16),
                preferred_element_type=jnp.float32)
    h = h + b1_ref[...]
    h = _gelu_exact(h).astype(jnp.bfloat16)
    y = jnp.dot(h, w2_ref[...].astype(jnp.bfloat16),
                preferred_element_type=jnp.float32)
    yg = yg_ref[...]
    w0 = wts_ref[:, 0:1]
    w1 = wts_ref[:, 1:2]
    out_ref[...] = (y + b2_ref[...] + yg[:, :DIM] * w0
                    + yg[:, DIM:] * w1)


def kernel(x, router_W, router_b, W1, b1, W2, b2, sW1, sb1, sW2, sb2):
    B, S, dim = x.shape
    xf = x.reshape(N, dim)

    rw = jnp.zeros((dim, 128), jnp.float32).at[:, :E].set(router_W)
    rb = jnp.full((1, 128), NEG, jnp.float32).at[0, :E].set(router_b)
    TB = 1024
    wts, tops = pl.pallas_call(
        _router_body,
        grid=(N // TB,),
        in_specs=[
            pl.BlockSpec((TB, dim), lambda t: (t, 0)),
            pl.BlockSpec((dim, 128), lambda t: (0, 0)),
            pl.BlockSpec((1, 128), lambda t: (0, 0)),
        ],
        out_specs=[
            pl.BlockSpec((TB, EPAD), lambda t: (t, 0)),
            pl.BlockSpec((TB, EPAD), lambda t: (t, 0)),
        ],
        out_shape=[
            jax.ShapeDtypeStruct((N, EPAD), jnp.float32),
            jax.ShapeDtypeStruct((N, EPAD), jnp.int32),
        ],
    )(xf, rw, rb)

    ej = tops[:, :2].reshape(64, 128)

    p2d, blk2d = pl.pallas_call(
        _meta_body,
        out_shape=[
            jax.ShapeDtypeStruct((64, 128), jnp.int32),
            jax.ShapeDtypeStruct((8, 128), jnp.int32),
        ],
    )(ej)
    p = p2d.reshape(NA)
    blk_expert = blk2d[0, :NBLK]

    pidx = p.reshape(NW, TPW // SCCH, SCCH, 2).transpose(0, 1, 3, 2)
    pidx = pidx.reshape(NW, 2 * (TPW // SCCH), SCCH)
    mesh = plsc.VectorSubcoreMesh(core_axis_name="c", subcore_axis_name="s")
    xg = pl.kernel(
        _sc_scatter_body,
        mesh=mesh,
        out_type=jax.ShapeDtypeStruct((NSLOT, DIM), jnp.float32),
        scratch_types=[
            pltpu.VMEM((2 * (TPW // SCCH), SCCH), jnp.int32),
            pltpu.VMEM((SCCH, DIM), jnp.float32),
            pltpu.SemaphoreType.DMA,
        ],
    )(xf, pidx)

    b1r = b1[:, None]
    b2r = b2[:, None]
    y = pl.pallas_call(
        _group_body,
        grid_spec=pltpu.PrefetchScalarGridSpec(
            num_scalar_prefetch=1,
            grid=(NBLK,),
            in_specs=[
                pl.BlockSpec((BS, DIM), lambda b, s: (b, 0)),
                pl.BlockSpec((1, DIM, H), lambda b, s: (s[b], 0, 0)),
                pl.BlockSpec((1, 1, H), lambda b, s: (s[b], 0, 0)),
                pl.BlockSpec((1, H, DIM), lambda b, s: (s[b], 0, 0)),
                pl.BlockSpec((1, 1, DIM), lambda b, s: (s[b], 0, 0)),
            ],
            out_specs=pl.BlockSpec((BS, DIM), lambda b, s: (b, 0)),
        ),
        out_shape=jax.ShapeDtypeStruct((NSLOT, DIM), jnp.float32),
        compiler_params=pltpu.CompilerParams(
            vmem_limit_bytes=62 * 1024 * 1024),
    )(blk_expert, xg, W1, b1r, W2, b2r)

    gidx = p.reshape(NW, NA // NW // GCH, GCH)
    yg = pl.kernel(
        _sc_gather_body,
        mesh=mesh,
        out_type=jax.ShapeDtypeStruct((NA, DIM), jnp.float32),
        scratch_types=[
            pltpu.VMEM((NA // NW // GCH, GCH), jnp.int32),
            pltpu.VMEM((GCH, DIM), jnp.float32),
            pltpu.SemaphoreType.DMA,
        ],
    )(y, gidx)

    yg2 = yg.reshape(N, 2 * DIM)
    TBS = 512
    out = pl.pallas_call(
        _shared_body,
        grid=(N // TBS,),
        in_specs=[
            pl.BlockSpec((TBS, dim), lambda t: (t, 0)),
            pl.BlockSpec((dim, H), lambda t: (0, 0)),
            pl.BlockSpec((1, H), lambda t: (0, 0)),
            pl.BlockSpec((H, dim), lambda t: (0, 0)),
            pl.BlockSpec((1, dim), lambda t: (0, 0)),
            pl.BlockSpec((TBS, EPAD), lambda t: (t, 0)),
            pl.BlockSpec((TBS, 2 * DIM), lambda t: (t, 0)),
        ],
        out_specs=pl.BlockSpec((TBS, dim), lambda t: (t, 0)),
        out_shape=jax.ShapeDtypeStruct((N, dim), jnp.float32),
        compiler_params=pltpu.CompilerParams(
            vmem_limit_bytes=62 * 1024 * 1024),
    )(xf, sW1, sb1[None], sW2, sb2[None], wts, yg2)

    return out.reshape(B, S, dim)


# speedup vs baseline: 1.0769x; 1.0117x over previous
import functools

import jax
import jax.numpy as jnp
from jax import lax
from jax.experimental import pallas as pl
from jax.experimental.pallas import tpu as pltpu
from jax.experimental.pallas import tpu_sc as plsc

DIM = 1024
E = 8
H = 2048
N = 4096
NA = 2 * N
EPAD = 16
NEG = -1e30

BS = 512
NBLK = NA // BS + E
NSLOT = NBLK * BS

NW = 32
TPW = N // NW
SCCH = 32
GCH = 32


def _gelu_exact(u):
    return 0.5 * u * (1.0 + jax.lax.erf(u * (2.0 ** -0.5)))


def _router_body(x_ref, rw_ref, rb_ref, w_ref, i_ref):
    x = x_ref[...]
    logits = jnp.dot(x, rw_ref[...], preferred_element_type=jnp.float32)
    logits = logits + rb_ref[...]
    m = jnp.max(logits, axis=1, keepdims=True)
    ex = jnp.exp(logits - m)
    probs = ex / jnp.sum(ex, axis=1, keepdims=True)
    lane = jax.lax.broadcasted_iota(jnp.int32, probs.shape, 1)
    m1 = jnp.max(probs, axis=1, keepdims=True)
    i1 = jnp.min(jnp.where(probs == m1, lane, 512), axis=1, keepdims=True)
    probs2 = jnp.where(lane == i1, NEG, probs)
    m2 = jnp.max(probs2, axis=1, keepdims=True)
    i2 = jnp.min(jnp.where(probs2 == m2, lane, 512), axis=1, keepdims=True)
    w = jnp.where(lane == 0, m1, 0.0) + jnp.where(lane == 1, m2, 0.0)
    ii = jnp.where(lane == 0, i1, 0) + jnp.where(lane == 1, i2, 0)
    w_ref[...] = w[:, :EPAD]
    i_ref[...] = ii[:, :EPAD]


def _meta_body(ej_ref, p_ref, blk_ref):
    ej = ej_ref[...]
    rows, cols = ej.shape
    r1 = jax.lax.broadcasted_iota(jnp.int32, (cols, cols), 0)
    c1 = jax.lax.broadcasted_iota(jnp.int32, (cols, cols), 1)
    U = (r1 <= c1).astype(jnp.float32)
    r2 = jax.lax.broadcasted_iota(jnp.int32, (rows, rows), 0)
    c2 = jax.lax.broadcasted_iota(jnp.int32, (rows, rows), 1)
    S = (c2 < r2).astype(jnp.float32)

    start = jnp.zeros((), jnp.int32)
    p = jnp.zeros(ej.shape, jnp.int32)
    bi = jax.lax.broadcasted_iota(jnp.int32, (8, 128), 1) * BS
    blk = jnp.zeros((8, 128), jnp.int32)
    for e in range(E):
        m = (ej == e).astype(jnp.float32)
        intra = jnp.dot(m, U, preferred_element_type=jnp.float32)
        rowtot = intra[:, cols - 1:cols]
        rowoff = jnp.dot(S, rowtot, preferred_element_type=jnp.float32)
        rank_excl = (intra + rowoff - 1.0).astype(jnp.int32)
        p = p + jnp.where(ej == e, start + rank_excl, 0)
        cnt = jnp.sum(m).astype(jnp.int32)
        cap = ((cnt + BS - 1) // BS) * BS
        blk = blk + jnp.where(
            jnp.logical_and(bi >= start, bi < start + cap), e, 0)
        start = start + cap
    blk = blk + jnp.where(bi >= start, E - 1, 0)
    p_ref[...] = p
    blk_ref[...] = blk


def _sc_scatter_body(x_hbm, pidx_hbm, xg_hbm, idx_v, rows0, rows1,
                     sem_i0, sem_i1, sem_o):
    wid = lax.axis_index("s") * 2 + lax.axis_index("c")
    nch = TPW // SCCH
    bufs = (rows0, rows1)
    sems = (sem_i0, sem_i1)
    pltpu.sync_copy(pidx_hbm.at[wid], idx_v)
    cps = [None] * nch
    cps[0] = pltpu.async_copy(
        x_hbm.at[pl.ds(wid * TPW, SCCH)], rows0, sem_i0)
    cps[1] = pltpu.async_copy(
        x_hbm.at[pl.ds(wid * TPW + SCCH, SCCH)], rows1, sem_i1)
    for c in range(nch):
        buf = bufs[c % 2]
        cps[c].wait()
        cp0 = pltpu.async_copy(buf, xg_hbm.at[idx_v.at[2 * c]], sem_o)
        cp1 = pltpu.async_copy(buf, xg_hbm.at[idx_v.at[2 * c + 1]], sem_o)
        cp0.wait()
        cp1.wait()
        if c + 2 < nch:
            cps[c + 2] = pltpu.async_copy(
                x_hbm.at[pl.ds(wid * TPW + (c + 2) * SCCH, SCCH)],
                bufs[c % 2], sems[c % 2])


def _group_body(blk_ref, xg_ref, w1_ref, b1_ref, w2_ref, b2_ref, y_ref):
    x = xg_ref[...].astype(jnp.bfloat16)
    h = jnp.dot(x, w1_ref[0].astype(jnp.bfloat16),
                preferred_element_type=jnp.float32)
    h = h + b1_ref[0]
    h = _gelu_exact(h).astype(jnp.bfloat16)
    y = jnp.dot(h, w2_ref[0].astype(jnp.bfloat16),
                preferred_element_type=jnp.float32)
    y_ref[...] = y + b2_ref[0]


def _sc_gather_body(y_hbm, gidx_hbm, yg_hbm, idx_v, rows0, rows1,
                    sem_i0, sem_i1):
    wid = lax.axis_index("s") * 2 + lax.axis_index("c")
    nch = NA // NW // GCH
    bufs = (rows0, rows1)
    sems = (sem_i0, sem_i1)
    pltpu.sync_copy(gidx_hbm.at[wid], idx_v)
    cps = [None] * nch
    cps[0] = pltpu.async_copy(y_hbm.at[idx_v.at[0]], rows0, sem_i0)
    cps[1] = pltpu.async_copy(y_hbm.at[idx_v.at[1]], rows1, sem_i1)
    for c in range(nch):
        cps[c].wait()
        pltpu.sync_copy(bufs[c % 2],
                        yg_hbm.at[pl.ds(wid * (NA // NW) + c * GCH, GCH)])
        if c + 2 < nch:
            cps[c + 2] = pltpu.async_copy(
                y_hbm.at[idx_v.at[c + 2]], bufs[c % 2], sems[c % 2])


def _shared_body(x_ref, w1_ref, b1_ref, w2_ref, b2_ref, wts_ref, yg_ref,
                 out_ref):
    x = x_ref[...].astype(jnp.bfloat16)
    h = jnp.dot(x, w1_ref[...].astype(jnp.bfloat16),
                preferred_element_type=jnp.float32)
    h = h + b1_ref[...]
    h = _gelu_exact(h).astype(jnp.bfloat16)
    y = jnp.dot(h, w2_ref[...].astype(jnp.bfloat16),
                preferred_element_type=jnp.float32)
    yg = yg_ref[...]
    w0 = wts_ref[:, 0:1]
    w1 = wts_ref[:, 1:2]
    out_ref[...] = (y + b2_ref[...] + yg[:, :DIM] * w0
                    + yg[:, DIM:] * w1)


def kernel(x, router_W, router_b, W1, b1, W2, b2, sW1, sb1, sW2, sb2):
    B, S, dim = x.shape
    xf = x.reshape(N, dim)

    rw = jnp.zeros((dim, 128), jnp.float32).at[:, :E].set(router_W)
    rb = jnp.full((1, 128), NEG, jnp.float32).at[0, :E].set(router_b)
    TB = 1024
    wts, tops = pl.pallas_call(
        _router_body,
        grid=(N // TB,),
        in_specs=[
            pl.BlockSpec((TB, dim), lambda t: (t, 0)),
            pl.BlockSpec((dim, 128), lambda t: (0, 0)),
            pl.BlockSpec((1, 128), lambda t: (0, 0)),
        ],
        out_specs=[
            pl.BlockSpec((TB, EPAD), lambda t: (t, 0)),
            pl.BlockSpec((TB, EPAD), lambda t: (t, 0)),
        ],
        out_shape=[
            jax.ShapeDtypeStruct((N, EPAD), jnp.float32),
            jax.ShapeDtypeStruct((N, EPAD), jnp.int32),
        ],
    )(xf, rw, rb)

    ej = tops[:, :2].reshape(64, 128)

    p2d, blk2d = pl.pallas_call(
        _meta_body,
        out_shape=[
            jax.ShapeDtypeStruct((64, 128), jnp.int32),
            jax.ShapeDtypeStruct((8, 128), jnp.int32),
        ],
    )(ej)
    p = p2d.reshape(NA)
    blk_expert = blk2d[0, :NBLK]

    pidx = p.reshape(NW, TPW // SCCH, SCCH, 2).transpose(0, 1, 3, 2)
    pidx = pidx.reshape(NW, 2 * (TPW // SCCH), SCCH)
    mesh = plsc.VectorSubcoreMesh(core_axis_name="c", subcore_axis_name="s")
    xg = pl.kernel(
        _sc_scatter_body,
        mesh=mesh,
        out_type=jax.ShapeDtypeStruct((NSLOT, DIM), jnp.float32),
        scratch_types=[
            pltpu.VMEM((2 * (TPW // SCCH), SCCH), jnp.int32),
            pltpu.VMEM((SCCH, DIM), jnp.float32),
            pltpu.VMEM((SCCH, DIM), jnp.float32),
            pltpu.SemaphoreType.DMA,
            pltpu.SemaphoreType.DMA,
            pltpu.SemaphoreType.DMA,
        ],
    )(xf, pidx)

    b1r = b1[:, None]
    b2r = b2[:, None]
    y = pl.pallas_call(
        _group_body,
        grid_spec=pltpu.PrefetchScalarGridSpec(
            num_scalar_prefetch=1,
            grid=(NBLK,),
            in_specs=[
                pl.BlockSpec((BS, DIM), lambda b, s: (b, 0)),
                pl.BlockSpec((1, DIM, H), lambda b, s: (s[b], 0, 0)),
                pl.BlockSpec((1, 1, H), lambda b, s: (s[b], 0, 0)),
                pl.BlockSpec((1, H, DIM), lambda b, s: (s[b], 0, 0)),
                pl.BlockSpec((1, 1, DIM), lambda b, s: (s[b], 0, 0)),
            ],
            out_specs=pl.BlockSpec((BS, DIM), lambda b, s: (b, 0)),
        ),
        out_shape=jax.ShapeDtypeStruct((NSLOT, DIM), jnp.float32),
        compiler_params=pltpu.CompilerParams(
            vmem_limit_bytes=62 * 1024 * 1024),
    )(blk_expert, xg, W1, b1r, W2, b2r)

    gidx = p.reshape(NW, NA // NW // GCH, GCH)
    yg = pl.kernel(
        _sc_gather_body,
        mesh=mesh,
        out_type=jax.ShapeDtypeStruct((NA, DIM), jnp.float32),
        scratch_types=[
            pltpu.VMEM((NA // NW // GCH, GCH), jnp.int32),
            pltpu.VMEM((GCH, DIM), jnp.float32),
            pltpu.VMEM((GCH, DIM), jnp.float32),
            pltpu.SemaphoreType.DMA,
            pltpu.SemaphoreType.DMA,
        ],
    )(y, gidx)

    yg2 = yg.reshape(N, 2 * DIM)
    TBS = 512
    out = pl.pallas_call(
        _shared_body,
        grid=(N // TBS,),
        in_specs=[
            pl.BlockSpec((TBS, dim), lambda t: (t, 0)),
            pl.BlockSpec((dim, H), lambda t: (0, 0)),
            pl.BlockSpec((1, H), lambda t: (0, 0)),
            pl.BlockSpec((H, dim), lambda t: (0, 0)),
            pl.BlockSpec((1, dim), lambda t: (0, 0)),
            pl.BlockSpec((TBS, EPAD), lambda t: (t, 0)),
            pl.BlockSpec((TBS, 2 * DIM), lambda t: (t, 0)),
        ],
        out_specs=pl.BlockSpec((TBS, dim), lambda t: (t, 0)),
        out_shape=jax.ShapeDtypeStruct((N, dim), jnp.float32),
        compiler_params=pltpu.CompilerParams(
            vmem_limit_bytes=62 * 1024 * 1024),
    )(xf, sW1, sb1[None], sW2, sb2[None], wts, yg2)

    return out.reshape(B, S, dim)
